# Initial kernel scaffold; baseline (speedup 1.0000x reference)
import jax
import jax.numpy as jnp
from jax.experimental import pallas as pl


def kernel(obs, table):
    raise NotImplementedError("write your pallas kernel here")


# speedup vs baseline: 11.4036x; 11.4036x over previous
import functools

import jax
import jax.numpy as jnp
from jax import lax
from jax.experimental import pallas as pl
from jax.experimental.pallas import tpu as pltpu
from jax.experimental.pallas import tpu_sc as plsc

N_CAT = 100000
EMBED_DIM = 64
BATCH = 16384
FIELDS = 100

_INFO = plsc.get_sparse_core_info()
NC, NS = _INFO.num_cores, _INFO.num_subcores
NW = NC * NS

IDX_W = 128
TOTAL = BATCH * FIELDS
ROWS = TOTAL // IDX_W
ROWS_PER_W = ROWS // NW
NSTREAM = 8
GROUPS = ROWS_PER_W // NSTREAM
GROUP_IDX = NSTREAM * IDX_W


def _body(obs_hbm, table_hbm, out_hbm, idx_v, rows_v, sem):
  wid = lax.axis_index("s") * NC + lax.axis_index("c")

  def group(g, carry):
    row0 = wid * ROWS_PER_W + g * NSTREAM
    pltpu.sync_copy(obs_hbm.at[pl.ds(row0, NSTREAM)], idx_v)
    copies = []
    for j in range(NSTREAM):
      copies.append(
          pltpu.async_copy(
              table_hbm.at[idx_v.at[j]],
              rows_v.at[pl.ds(j * IDX_W, IDX_W)],
              sem,
          )
      )
    for cp in copies:
      cp.wait()
    pltpu.sync_copy(rows_v, out_hbm.at[pl.ds(row0 * IDX_W, GROUP_IDX)])
    return carry

  lax.fori_loop(0, GROUPS, group, 0)


@jax.jit
def kernel(obs, table):
  idx = obs.reshape(ROWS, IDX_W).astype(jnp.int32)
  mesh = plsc.VectorSubcoreMesh(core_axis_name="c", subcore_axis_name="s")
  out = pl.kernel(
      _body,
      out_type=jax.ShapeDtypeStruct((TOTAL, EMBED_DIM), jnp.float32),
      mesh=mesh,
      scratch_types=[
          pltpu.VMEM((NSTREAM, IDX_W), jnp.int32),
          pltpu.VMEM((GROUP_IDX, EMBED_DIM), jnp.float32),
          pltpu.SemaphoreType.DMA,
      ],
      compiler_params=pltpu.CompilerParams(use_tc_tiling_on_sc=False),
  )(idx, table)
  return out.reshape(BATCH, FIELDS, EMBED_DIM)


# speedup vs baseline: 11.7588x; 1.0311x over previous
import functools

import jax
import jax.numpy as jnp
from jax import lax
from jax.experimental import pallas as pl
from jax.experimental.pallas import tpu as pltpu
from jax.experimental.pallas import tpu_sc as plsc

N_CAT = 100000
EMBED_DIM = 64
BATCH = 16384
FIELDS = 100

_INFO = plsc.get_sparse_core_info()
NC, NS = _INFO.num_cores, _INFO.num_subcores
NW = NC * NS

IDX_W = 128
TOTAL = BATCH * FIELDS
ROWS = TOTAL // IDX_W
ROWS_PER_W = ROWS // NW
NSTREAM = 4
GROUPS = ROWS_PER_W // NSTREAM
PAIRS = GROUPS // 2
GROUP_IDX = NSTREAM * IDX_W


def _body(obs_hbm, table_hbm, out_hbm,
          idx0, idx1, rows0, rows1, gsem, isem, wsem0, wsem1):
  wid = lax.axis_index("s") * NC + lax.axis_index("c")
  row_base = wid * ROWS_PER_W
  idx_bufs = (idx0, idx1)
  rows_bufs = (rows0, rows1)
  wsems = (wsem0, wsem1)

  def idx_src(g):
    return obs_hbm.at[pl.ds(row_base + g * NSTREAM, NSTREAM)]

  pltpu.async_copy(idx_src(0), idx0, isem)

  def pair(p, carry):
    for b in (0, 1):
      g = 2 * p + b
      idx_v, rows_v, wsem = idx_bufs[b], rows_bufs[b], wsems[b]
      @pl.when(p > 0)
      def _():
        pltpu.make_async_copy(
            rows_v, out_hbm.at[pl.ds(0, GROUP_IDX)], wsem).wait()
      pltpu.make_async_copy(idx_src(g), idx_v, isem).wait()
      copies = [
          pltpu.async_copy(
              table_hbm.at[idx_v.at[j]],
              rows_v.at[pl.ds(j * IDX_W, IDX_W)],
              gsem,
          )
          for j in range(NSTREAM)
      ]
      @pl.when(g + 1 < GROUPS)
      def _():
        pltpu.async_copy(idx_src(g + 1), idx_bufs[1 - b], isem)
      for cp in copies:
        cp.wait()
      pltpu.async_copy(
          rows_v,
          out_hbm.at[pl.ds((row_base + g * NSTREAM) * IDX_W, GROUP_IDX)],
          wsem,
      )
    return carry

  lax.fori_loop(0, PAIRS, pair, 0)

  for b in (0, 1):
    pltpu.make_async_copy(
        rows_bufs[b], out_hbm.at[pl.ds(0, GROUP_IDX)], wsems[b]).wait()


@jax.jit
def kernel(obs, table):
  idx = obs.reshape(ROWS, IDX_W).astype(jnp.int32)
  mesh = plsc.VectorSubcoreMesh(core_axis_name="c", subcore_axis_name="s")
  out = pl.kernel(
      _body,
      out_type=jax.ShapeDtypeStruct((TOTAL, EMBED_DIM), jnp.float32),
      mesh=mesh,
      scratch_types=[
          pltpu.VMEM((NSTREAM, IDX_W), jnp.int32),
          pltpu.VMEM((NSTREAM, IDX_W), jnp.int32),
          pltpu.VMEM((GROUP_IDX, EMBED_DIM), jnp.float32),
          pltpu.VMEM((GROUP_IDX, EMBED_DIM), jnp.float32),
          pltpu.SemaphoreType.DMA,
          pltpu.SemaphoreType.DMA,
          pltpu.SemaphoreType.DMA,
          pltpu.SemaphoreType.DMA,
      ],
      compiler_params=pltpu.CompilerParams(use_tc_tiling_on_sc=False),
  )(idx, table)
  return out.reshape(BATCH, FIELDS, EMBED_DIM)


# speedup vs baseline: 11.7832x; 1.0021x over previous
import functools

import jax
import jax.numpy as jnp
from jax import lax
from jax.experimental import pallas as pl
from jax.experimental.pallas import tpu as pltpu
from jax.experimental.pallas import tpu_sc as plsc

N_CAT = 100000
EMBED_DIM = 64
BATCH = 16384
FIELDS = 100

_INFO = plsc.get_sparse_core_info()
NC, NS = _INFO.num_cores, _INFO.num_subcores
NW = NC * NS

TOTAL = BATCH * FIELDS
PER_W = TOTAL // NW
GROUP = 512
GROUPS = PER_W // GROUP
PAIRS = GROUPS // 2


def _body(obs_hbm, table_hbm, out_hbm,
          idx0, idx1, rows0, rows1, gsem, isem, wsem0, wsem1):
  wid = lax.axis_index("s") * NC + lax.axis_index("c")
  base = wid * PER_W
  idx_bufs = (idx0, idx1)
  rows_bufs = (rows0, rows1)
  wsems = (wsem0, wsem1)

  def idx_src(g):
    return obs_hbm.at[pl.ds(base + g * GROUP, GROUP)]

  def out_dst(g):
    return out_hbm.at[pl.ds(base + g * GROUP, GROUP)]

  pltpu.async_copy(idx_src(0), idx0, isem)

  def pair(p, carry):
    for b in (0, 1):
      g = 2 * p + b
      idx_v, rows_v, wsem = idx_bufs[b], rows_bufs[b], wsems[b]
      @pl.when(p > 0)
      def _():
        pltpu.make_async_copy(rows_v, out_dst(0), wsem).wait()
      pltpu.make_async_copy(idx_src(g), idx_v, isem).wait()
      gather = pltpu.async_copy(table_hbm.at[idx_v], rows_v, gsem)
      @pl.when(g + 1 < GROUPS)
      def _():
        pltpu.async_copy(idx_src(g + 1), idx_bufs[1 - b], isem)
      gather.wait()
      pltpu.async_copy(rows_v, out_dst(g), wsem)
    return carry

  lax.fori_loop(0, PAIRS, pair, 0)

  for b in (0, 1):
    pltpu.make_async_copy(rows_bufs[b], out_dst(0), wsems[b]).wait()


@jax.jit
def kernel(obs, table):
  idx = obs.reshape(TOTAL).astype(jnp.int32)
  mesh = plsc.VectorSubcoreMesh(core_axis_name="c", subcore_axis_name="s")
  out = pl.kernel(
      _body,
      out_type=jax.ShapeDtypeStruct((TOTAL, EMBED_DIM), jnp.float32),
      mesh=mesh,
      scratch_types=[
          pltpu.VMEM((GROUP,), jnp.int32),
          pltpu.VMEM((GROUP,), jnp.int32),
          pltpu.VMEM((GROUP, EMBED_DIM), jnp.float32),
          pltpu.VMEM((GROUP, EMBED_DIM), jnp.float32),
          pltpu.SemaphoreType.DMA,
          pltpu.SemaphoreType.DMA,
          pltpu.SemaphoreType.DMA,
          pltpu.SemaphoreType.DMA,
      ],
      compiler_params=pltpu.CompilerParams(use_tc_tiling_on_sc=False),
  )(idx, table)
  return out.reshape(BATCH, FIELDS, EMBED_DIM)
